# Initial kernel scaffold; baseline (speedup 1.0000x reference)
import jax
import jax.numpy as jnp
from jax.experimental import pallas as pl


def kernel(xyz, points, w0, b0, g0, beta0, w1, b1, g1, beta1):
    raise NotImplementedError("write your pallas kernel here")


# speedup vs baseline: 13.1236x; 13.1236x over previous
import functools

import jax
import jax.numpy as jnp
from jax import lax
from jax.experimental import pallas as pl
from jax.experimental.pallas import tpu as pltpu
from jax.experimental.pallas import tpu_sc as plsc

B, N, S, K = 4, 8192, 2048, 16
CIN, CH = 35, 64
CHP = 128
TS = 256
CHN = 1024
NCH = N // CHN
BIG = 1e30
R = B * S * K
GCH = 512


def _fps_body(xyzt_ref, xc_ref, yc_ref, zc_ref, fidx_ref, dist_ref):
    col3 = lax.broadcasted_iota(jnp.int32, (3, B, N), 2)
    col2 = lax.broadcasted_iota(jnp.int32, (B, N), 1)
    xyzt = xyzt_ref[...]
    dist_ref[...] = jnp.full((B, N), 1e10, dtype=jnp.float32)

    def body(i, f):
        masked = jnp.where(col3 == f[None, :, :], xyzt, 0.0)
        cen = jnp.sum(masked, axis=2)
        xc_ref[pl.ds(i, 1), :] = cen[0:1]
        yc_ref[pl.ds(i, 1), :] = cen[1:2]
        zc_ref[pl.ds(i, 1), :] = cen[2:3]
        fidx_ref[pl.ds(i, 1), :] = jnp.transpose(f)
        d = jnp.sum((xyzt - cen[:, :, None]) ** 2, axis=0)
        dist = jnp.minimum(dist_ref[...], d)
        dist_ref[...] = dist
        m = jnp.max(dist, axis=1, keepdims=True)
        am = jnp.min(jnp.where(dist == m, col2, N), axis=1, keepdims=True)
        return am.astype(jnp.int32)

    lax.fori_loop(0, S, body, jnp.zeros((B, 1), jnp.int32))


def _fps(xyzt):
    return pl.pallas_call(
        _fps_body,
        out_shape=[jax.ShapeDtypeStruct((S, B), jnp.float32)] * 3
        + [jax.ShapeDtypeStruct((S, B), jnp.int32)],
        scratch_shapes=[pltpu.VMEM((B, N), jnp.float32)],
    )(xyzt)


def _knn_body(xyzc_ref, xc_ref, yc_ref, zc_ref, idx_ref, d_ref):
    b = pl.program_id(0)
    cx = jnp.transpose(xc_ref[0])
    cy = jnp.transpose(yc_ref[0])
    cz = jnp.transpose(zc_ref[0])
    csq = cx * cx + cy * cy + cz * cz

    ct = jnp.concatenate([cx, cy, cz], axis=1)

    def fill(c, _):
        xs = xyzc_ref[0, c]
        x0, y0, z0 = xs[0:1], xs[1:2], xs[2:3]
        xsq = x0 * x0 + y0 * y0 + z0 * z0
        cross = jnp.dot(ct, xs, preferred_element_type=jnp.float32)
        d_ref[c] = (csq + xsq) - 2.0 * cross
        return 0

    lax.fori_loop(0, NCH, fill, 0)

    colc = lax.broadcasted_iota(jnp.int32, (TS, CHN), 1)

    for k in range(K):
        def scan_chunk(c, carry):
            m, am = carry
            dc = d_ref[c]
            mc = jnp.min(dc, axis=1, keepdims=True)
            amc = jnp.min(jnp.where(dc == mc, colc, CHN),
                          axis=1, keepdims=True) + c * CHN
            take = mc < m
            return jnp.where(take, mc, m), jnp.where(take, amc, am)

        m0 = jnp.full((TS, 1), 2e30, jnp.float32)
        a0 = jnp.zeros((TS, 1), jnp.int32)
        m, am = lax.fori_loop(0, NCH, scan_chunk, (m0, a0))
        idx_ref[0, k:k + 1, :] = jnp.transpose(am + b * N)

        def mask(c, _):
            d_ref[c] = jnp.where(colc + c * CHN == am, BIG, d_ref[c])
            return 0

        lax.fori_loop(0, NCH, mask, 0)


def _knn(xyzc, xc3, yc3, zc3):
    return pl.pallas_call(
        _knn_body,
        grid=(B, S // TS),
        in_specs=[
            pl.BlockSpec((1, NCH, 3, CHN), lambda b, st: (b, 0, 0, 0)),
            pl.BlockSpec((1, 1, TS), lambda b, st: (b, 0, st)),
            pl.BlockSpec((1, 1, TS), lambda b, st: (b, 0, st)),
            pl.BlockSpec((1, 1, TS), lambda b, st: (b, 0, st)),
        ],
        out_specs=pl.BlockSpec((1, K, TS), lambda b, st: (b, 0, st)),
        out_shape=jax.ShapeDtypeStruct((B, K, S), jnp.int32),
        scratch_shapes=[pltpu.VMEM((NCH, TS, CHN), jnp.float32)],
    )(xyzc, xc3, yc3, zc3)


def _proj_body(cat_ref, w0t_ref, u_ref):
    u_ref[0] = jnp.dot(cat_ref[0], w0t_ref[...],
                       preferred_element_type=jnp.float32,
                       precision=lax.Precision.HIGHEST)


def _proj(cat, w0t):
    return pl.pallas_call(
        _proj_body,
        grid=(B,),
        in_specs=[
            pl.BlockSpec((1, N, CIN), lambda b: (b, 0, 0)),
            pl.BlockSpec((CIN, CHP), lambda b: (0, 0)),
        ],
        out_specs=pl.BlockSpec((1, N, CHP), lambda b: (b, 0, 0)),
        out_shape=jax.ShapeDtypeStruct((B, N, CHP), jnp.float32),
    )(cat, w0t)


def _sc_gather(table, idxf):
    info = plsc.get_sparse_core_info()
    nw = info.num_cores * info.num_subcores
    rows_per_w = R // nw
    nch = rows_per_w // GCH
    mesh = plsc.VectorSubcoreMesh(core_axis_name="c", subcore_axis_name="s")

    @functools.partial(
        pl.kernel, mesh=mesh,
        out_type=jax.ShapeDtypeStruct((R, CHP), jnp.float32),
        scratch_types=[
            pltpu.VMEM((GCH,), jnp.int32),
            pltpu.VMEM((GCH, CHP), jnp.float32),
            pltpu.SemaphoreType.DMA,
        ],
    )
    def k(table_hbm, idx_hbm, out_hbm, idx_v, rows_v, sem):
        wid = lax.axis_index("s") * info.num_cores + lax.axis_index("c")
        base = wid * rows_per_w
        for c in range(nch):
            off = base + c * GCH
            pltpu.sync_copy(idx_hbm.at[pl.ds(off, GCH)], idx_v)
            pltpu.async_copy(table_hbm.at[idx_v], rows_v, sem).wait()
            pltpu.sync_copy(rows_v, out_hbm.at[pl.ds(off, GCH)])

    return k(table, idxf)


def _z1(g_ref, xc_ref, yc_ref, zc_ref, w0_ref, b0_ref):
    cx = jnp.transpose(xc_ref[0])
    cy = jnp.transpose(yc_ref[0])
    cz = jnp.transpose(zc_ref[0])
    w0x = jnp.transpose(w0_ref[:, 0:3])
    v = b0_ref[...] - (cx * w0x[0:1] + cy * w0x[1:2] + cz * w0x[2:3])
    return g_ref[0][:, :, 0:CH] + v[None]


def _acc_stats(stats_ref, z2d):
    first = (pl.program_id(0) == 0) & (pl.program_id(1) == 0)

    @pl.when(first)
    def _():
        stats_ref[...] = jnp.zeros_like(stats_ref)

    stats_ref[0:1, :] += jnp.sum(z2d, axis=0, keepdims=True)
    stats_ref[1:2, :] += jnp.sum(z2d * z2d, axis=0, keepdims=True)


def _stats1_body(g_ref, xc_ref, yc_ref, zc_ref, w0_ref, b0_ref, stats_ref):
    z1 = _z1(g_ref, xc_ref, yc_ref, zc_ref, w0_ref, b0_ref)
    _acc_stats(stats_ref, z1.reshape(K * TS, CH))


def _z2(g_ref, xc_ref, yc_ref, zc_ref, w0_ref, b0_ref, s1_ref, t1_ref,
        w1t_ref, b1_ref):
    z1 = _z1(g_ref, xc_ref, yc_ref, zc_ref, w0_ref, b0_ref)
    a1 = jnp.maximum(z1 * s1_ref[...][None] + t1_ref[...][None], 0.0)
    return jnp.dot(a1.reshape(K * TS, CH), w1t_ref[...],
                   preferred_element_type=jnp.float32,
                   precision=lax.Precision.HIGHEST) + b1_ref[...]


def _stats2_body(g_ref, xc_ref, yc_ref, zc_ref, w0_ref, b0_ref, s1_ref,
                 t1_ref, w1t_ref, b1_ref, stats_ref):
    _acc_stats(stats_ref, _z2(g_ref, xc_ref, yc_ref, zc_ref, w0_ref,
                              b0_ref, s1_ref, t1_ref, w1t_ref, b1_ref))


def _final_body(g_ref, xc_ref, yc_ref, zc_ref, w0_ref, b0_ref, s1_ref,
                t1_ref, w1t_ref, b1_ref, s2_ref, t2_ref, out_ref):
    z2 = _z2(g_ref, xc_ref, yc_ref, zc_ref, w0_ref, b0_ref, s1_ref,
             t1_ref, w1t_ref, b1_ref)
    y = jnp.maximum(z2 * s2_ref[...] + t2_ref[...], 0.0)
    out_ref[0] = jnp.max(y.reshape(K, TS, CH), axis=0)


def _mlp_specs(extra):
    tile = [
        pl.BlockSpec((1, K, TS, CHP), lambda b, st: (b, 0, st, 0)),
        pl.BlockSpec((1, 1, TS), lambda b, st: (b, 0, st)),
        pl.BlockSpec((1, 1, TS), lambda b, st: (b, 0, st)),
        pl.BlockSpec((1, 1, TS), lambda b, st: (b, 0, st)),
        pl.BlockSpec((CH, CIN), lambda b, st: (0, 0)),
        pl.BlockSpec((1, CH), lambda b, st: (0, 0)),
    ]
    tile += [pl.BlockSpec(s, lambda b, st: (0, 0)) for s in extra]
    return tile


def _stats1(g4, xc, yc, zc, w0, b0r):
    return pl.pallas_call(
        _stats1_body,
        grid=(B, S // TS),
        in_specs=_mlp_specs([]),
        out_specs=pl.BlockSpec((8, CH), lambda b, st: (0, 0)),
        out_shape=jax.ShapeDtypeStruct((8, CH), jnp.float32),
    )(g4, xc, yc, zc, w0, b0r)


def _stats2(g4, xc, yc, zc, w0, b0r, s1, t1, w1t, b1r):
    return pl.pallas_call(
        _stats2_body,
        grid=(B, S // TS),
        in_specs=_mlp_specs([(1, CH), (1, CH), (CH, CH), (1, CH)]),
        out_specs=pl.BlockSpec((8, CH), lambda b, st: (0, 0)),
        out_shape=jax.ShapeDtypeStruct((8, CH), jnp.float32),
    )(g4, xc, yc, zc, w0, b0r, s1, t1, w1t, b1r)


def _final(g4, xc, yc, zc, w0, b0r, s1, t1, w1t, b1r, s2, t2):
    return pl.pallas_call(
        _final_body,
        grid=(B, S // TS),
        in_specs=_mlp_specs([(1, CH), (1, CH), (CH, CH), (1, CH),
                             (1, CH), (1, CH)]),
        out_specs=pl.BlockSpec((1, TS, CH), lambda b, st: (b, st, 0)),
        out_shape=jax.ShapeDtypeStruct((B, S, CH), jnp.float32),
    )(g4, xc, yc, zc, w0, b0r, s1, t1, w1t, b1r, s2, t2)


def _affine(stats, g, beta, cnt):
    mean = stats[0] / cnt
    var = stats[1] / cnt - mean * mean
    scale = g / jnp.sqrt(var + 1e-5)
    return (scale.reshape(1, CH),
            (beta - mean * scale).reshape(1, CH))


def kernel(xyz, points, w0, b0, g0, beta0, w1, b1, g1, beta1):
    xyzt = jnp.transpose(xyz, (2, 0, 1))
    xcs, ycs, zcs, _ = _fps(xyzt)
    new_xyz = jnp.transpose(jnp.stack([xcs, ycs, zcs], axis=-1), (1, 0, 2))
    xc = xcs.T.reshape(B, 1, S)
    yc = ycs.T.reshape(B, 1, S)
    zc = zcs.T.reshape(B, 1, S)

    xyzc = jnp.transpose(
        jnp.transpose(xyz, (0, 2, 1)).reshape(B, 3, NCH, CHN), (0, 2, 1, 3))
    idx = _knn(xyzc, xc, yc, zc)
    cat = jnp.concatenate([xyz, points], axis=-1)
    w0p = jnp.concatenate([w0.T, jnp.zeros((CIN, CHP - CH), w0.dtype)], axis=1)
    u = _proj(cat, w0p)
    g = _sc_gather(u.reshape(B * N, CHP), idx.reshape(R))
    g4 = g.reshape(B, K, S, CHP)

    b0r = b0.reshape(1, CH)
    b1r = b1.reshape(1, CH)
    cnt = jnp.float32(B * S * K)
    st1 = _stats1(g4, xc, yc, zc, w0, b0r)
    s1, t1 = _affine(st1, g0, beta0, cnt)
    st2 = _stats2(g4, xc, yc, zc, w0, b0r, s1, t1, w1.T, b1r)
    s2, t2 = _affine(st2, g1, beta1, cnt)
    out = _final(g4, xc, yc, zc, w0, b0r, s1, t1, w1.T, b1r, s2, t2)
    return new_xyz, out
